# baseline (speedup 1.0000x reference)
import functools

import jax
import jax.numpy as jnp
from jax import lax
from jax.experimental import pallas as pl
from jax.experimental.pallas import tpu as pltpu, tpu_sc as plsc

NUM_EMB = 1000000
DIM = 64
BATCH = 16384

_info = plsc.get_sparse_core_info()
_NC, _NS = _info.num_cores, _info.num_subcores
_NW = _NC * _NS

SPLIT = 8704
_BPW = SPLIT // _NW
_TC_N = BATCH - SPLIT


def _sc_body(idx_hbm, table_hbm, out_hbm, idx_v, rows_v, sem):
    wid = lax.axis_index("s") * _NC + lax.axis_index("c")
    base = wid * _BPW
    pltpu.sync_copy(idx_hbm.at[pl.ds(base, _BPW)], idx_v)

    def fire(c, carry):
        b = c * 16
        v = lax.rem(idx_v[pl.ds(b, 16)], jnp.full((16,), NUM_EMB, jnp.int32))
        for j in range(16):
            pltpu.async_copy(table_hbm.at[pl.ds(v[j], 1), :],
                             rows_v.at[pl.ds(b + j, 1), :], sem)
        return carry

    lax.fori_loop(0, _BPW // 16, fire, 0)
    pltpu.make_async_copy(table_hbm.at[pl.ds(0, _BPW), :], rows_v, sem).wait()
    pltpu.sync_copy(rows_v, out_hbm.at[pl.ds(base, _BPW)])


def _tc_body(idx_s, table_hbm, out_hbm, rows_v, sem, sem2):
    def fire(i, carry):
        r = lax.rem(idx_s[i], NUM_EMB)
        pltpu.make_async_copy(table_hbm.at[pl.ds(r, 1), :],
                              rows_v.at[pl.ds(i, 1), :], sem).start()
        return carry

    lax.fori_loop(0, _TC_N, fire, 0, unroll=16)
    pltpu.make_async_copy(table_hbm.at[pl.ds(0, _TC_N), :],
                          rows_v, sem).wait()
    out_copy = pltpu.make_async_copy(rows_v, out_hbm, sem2)
    out_copy.start()
    out_copy.wait()


@jax.jit
def _gather(idx, weight):
    mesh = plsc.VectorSubcoreMesh(core_axis_name="c", subcore_axis_name="s")
    sc_k = functools.partial(
        pl.kernel,
        mesh=mesh,
        out_type=jax.ShapeDtypeStruct((SPLIT, DIM), jnp.float32),
        scratch_types=[
            pltpu.VMEM((_BPW,), jnp.int32),
            pltpu.VMEM((_BPW, DIM), jnp.float32),
            pltpu.SemaphoreType.DMA,
        ],
    )(_sc_body)
    out_tc = pl.pallas_call(
        _tc_body,
        in_specs=[
            pl.BlockSpec(memory_space=pltpu.SMEM),
            pl.BlockSpec(memory_space=pltpu.MemorySpace.HBM),
        ],
        out_specs=pl.BlockSpec(memory_space=pltpu.MemorySpace.HBM),
        out_shape=jax.ShapeDtypeStruct((_TC_N, DIM), jnp.float32),
        scratch_shapes=[pltpu.VMEM((_TC_N, DIM), jnp.float32),
                        pltpu.SemaphoreType.DMA,
                        pltpu.SemaphoreType.DMA],
    )(idx[SPLIT:], weight)
    out_sc = sc_k(idx[:SPLIT], weight)

    return jnp.concatenate([out_sc, out_tc], axis=0)


def kernel(x, weight):
    idx = x.astype(jnp.int32)
    return _gather(idx, weight)


# speedup vs baseline: 1.0039x; 1.0039x over previous
import functools

import jax
from jax.experimental import compute_on as _compute_on
import jax.numpy as jnp
from jax import lax
from jax.experimental import pallas as pl
from jax.experimental.pallas import tpu as pltpu, tpu_sc as plsc

NUM_EMB = 1000000
DIM = 64
BATCH = 16384

_info = plsc.get_sparse_core_info()
_NC, _NS = _info.num_cores, _info.num_subcores
_NW = _NC * _NS

SPLIT = 8704
_BPW = SPLIT // _NW
_TC_N = BATCH - SPLIT


def _sc_body(idx_hbm, table_hbm, out_hbm, idx_v, rows_v, sem):
    wid = lax.axis_index("s") * _NC + lax.axis_index("c")
    base = wid * _BPW
    pltpu.sync_copy(idx_hbm.at[pl.ds(base, _BPW)], idx_v)

    def fire(c, carry):
        b = c * 16
        v = lax.rem(idx_v[pl.ds(b, 16)], jnp.full((16,), NUM_EMB, jnp.int32))
        for j in range(16):
            pltpu.async_copy(table_hbm.at[pl.ds(v[j], 1), :],
                             rows_v.at[pl.ds(b + j, 1), :], sem)
        return carry

    lax.fori_loop(0, _BPW // 16, fire, 0)
    pltpu.make_async_copy(table_hbm.at[pl.ds(0, _BPW), :], rows_v, sem).wait()
    pltpu.sync_copy(rows_v, out_hbm.at[pl.ds(base, _BPW)])


def _tc_body(idx_s, table_hbm, out_hbm, rows_v, sem, sem2):
    def fire(i, carry):
        r = lax.rem(idx_s[i], NUM_EMB)
        pltpu.make_async_copy(table_hbm.at[pl.ds(r, 1), :],
                              rows_v.at[pl.ds(i, 1), :], sem).start()
        return carry

    lax.fori_loop(0, _TC_N, fire, 0, unroll=16)
    pltpu.make_async_copy(table_hbm.at[pl.ds(0, _TC_N), :],
                          rows_v, sem).wait()
    out_copy = pltpu.make_async_copy(rows_v, out_hbm, sem2)
    out_copy.start()
    out_copy.wait()


@jax.jit
def _gather(idx, weight):
    mesh = plsc.VectorSubcoreMesh(core_axis_name="c", subcore_axis_name="s")
    sc_k = functools.partial(
        pl.kernel,
        mesh=mesh,
        out_type=jax.ShapeDtypeStruct((SPLIT, DIM), jnp.float32),
        scratch_types=[
            pltpu.VMEM((_BPW,), jnp.int32),
            pltpu.VMEM((_BPW, DIM), jnp.float32),
            pltpu.SemaphoreType.DMA,
        ],
    )(_sc_body)
    out_tc = pl.pallas_call(
        _tc_body,
        in_specs=[
            pl.BlockSpec(memory_space=pltpu.SMEM),
            pl.BlockSpec(memory_space=pltpu.MemorySpace.HBM),
        ],
        out_specs=pl.BlockSpec(memory_space=pltpu.MemorySpace.HBM),
        out_shape=jax.ShapeDtypeStruct((_TC_N, DIM), jnp.float32),
        scratch_shapes=[pltpu.VMEM((_TC_N, DIM), jnp.float32),
                        pltpu.SemaphoreType.DMA,
                        pltpu.SemaphoreType.DMA],
    )(idx[SPLIT:], weight)
    with _compute_on.compute_on('tpu_sparsecore'):
        out_sc = sc_k(idx[:SPLIT], weight)

    return jnp.concatenate([out_sc, out_tc], axis=0)


def kernel(x, weight):
    idx = x.astype(jnp.int32)
    return _gather(idx, weight)
